# baseline (speedup 1.0000x reference)
import functools

import jax
import jax.numpy as jnp
from jax import lax
from jax.experimental import pallas as pl
from jax.experimental.pallas import tpu as pltpu
from jax.experimental.pallas import tpu_sc as plsc

VOCAB = 100000
EMBED_DIM = 64
BATCH = 4096
SEQ_LEN = 200

NUM_CORES = 2
NUM_SUBCORES = 16
NUM_WORKERS = NUM_CORES * NUM_SUBCORES

FLAT = BATCH * SEQ_LEN
PER_W = FLAT // NUM_WORKERS
SEQS_PER_CHUNK = 4
CHUNK = SEQS_PER_CHUNK * SEQ_LEN
N_CHUNKS = PER_W // CHUNK


def _embed_body(x_hbm, tok_hbm, pos_hbm, out_hbm,
                idx0, idx1, rows0, rows1, gsem, wsem0, wsem1):
  cid = lax.axis_index("c")
  sid = lax.axis_index("s")
  wid = sid * NUM_CORES + cid
  base = wid * PER_W
  idx = (idx0, idx1)
  rows = (rows0, rows1)
  wsem = (wsem0, wsem1)

  @pl.loop(0, N_CHUNKS, step=2)
  def _chunk(ci):
    for b in range(2):
      i = ci + b
      off = base + i * CHUNK

      @pl.when(i >= 2)
      def _():
        pltpu.make_async_copy(
            rows[b], out_hbm.at[pl.ds(base, CHUNK)], wsem[b]).wait()

      pltpu.sync_copy(x_hbm.at[pl.ds(off, CHUNK)], idx[b])
      pltpu.sync_copy(pos_hbm, rows[b])
      pltpu.async_copy(tok_hbm.at[idx[b]], rows[b], gsem, add=True).wait()
      pltpu.async_copy(rows[b], out_hbm.at[pl.ds(off, CHUNK)], wsem[b])

  for b in range(2):
    pltpu.make_async_copy(
        rows[b], out_hbm.at[pl.ds(base, CHUNK)], wsem[b]).wait()


@jax.jit
def _embed(x_flat, token_table, pos_table):
  mesh = plsc.VectorSubcoreMesh(
      core_axis_name="c", subcore_axis_name="s",
      num_cores=NUM_CORES, num_subcores=NUM_SUBCORES,
  )
  run = pl.kernel(
      _embed_body,
      out_type=jax.ShapeDtypeStruct((FLAT, EMBED_DIM), jnp.float32),
      mesh=mesh,
      compiler_params=pltpu.CompilerParams(use_tc_tiling_on_sc=False),
      scratch_types=[
          pltpu.VMEM((CHUNK,), jnp.int32),
          pltpu.VMEM((CHUNK,), jnp.int32),
          pltpu.VMEM((CHUNK, EMBED_DIM), jnp.float32),
          pltpu.VMEM((CHUNK, EMBED_DIM), jnp.float32),
          pltpu.SemaphoreType.DMA,
          pltpu.SemaphoreType.DMA,
          pltpu.SemaphoreType.DMA,
      ],
  )
  return run(x_flat, token_table, pos_table)


def kernel(x, token_table, pos_table):
  x_flat = x.reshape(FLAT).astype(jnp.int32)
  pos_block = jnp.tile(pos_table, (SEQS_PER_CHUNK, 1))
  out = _embed(x_flat, token_table, pos_block)
  return out.reshape(BATCH, SEQ_LEN, EMBED_DIM)


# speedup vs baseline: 1.2977x; 1.2977x over previous
import functools

import jax
import jax.numpy as jnp
from jax import lax
from jax.experimental import pallas as pl
from jax.experimental.pallas import tpu as pltpu
from jax.experimental.pallas import tpu_sc as plsc

VOCAB = 100000
EMBED_DIM = 64
BATCH = 4096
SEQ_LEN = 200

NUM_CORES = 2
NUM_SUBCORES = 16
NUM_WORKERS = NUM_CORES * NUM_SUBCORES

FLAT = BATCH * SEQ_LEN
PER_W = FLAT // NUM_WORKERS
SEQS_PER_CHUNK = 4
CHUNK = SEQS_PER_CHUNK * SEQ_LEN
N_CHUNKS = PER_W // CHUNK
LANES = 16
D_REGS = EMBED_DIM // LANES


def _embed_body(x_hbm, tok_hbm, pos_hbm, out_hbm,
                idx0, idx1, rows0, rows1, pos_v,
                gsem0, gsem1, wsem0, wsem1):
  cid = lax.axis_index("c")
  sid = lax.axis_index("s")
  wid = sid * NUM_CORES + cid
  base = wid * PER_W
  idx = (idx0, idx1)
  rows = (rows0, rows1)
  gsem = (gsem0, gsem1)
  wsem = (wsem0, wsem1)

  pltpu.sync_copy(pos_hbm, pos_v)

  pltpu.sync_copy(x_hbm.at[pl.ds(base, CHUNK)], idx[0])
  pltpu.async_copy(tok_hbm.at[idx[0]], rows[0], gsem[0])

  @pl.loop(0, N_CHUNKS, step=2)
  def _chunk(ci):
    for b in range(2):
      i = ci + b
      nb = 1 - b

      @pl.when(i + 1 < N_CHUNKS)
      def _():
        @pl.when(i >= 1)
        def _():
          pltpu.make_async_copy(
              rows[nb], out_hbm.at[pl.ds(base, CHUNK)], wsem[nb]).wait()
        off_n = base + (i + 1) * CHUNK
        pltpu.sync_copy(x_hbm.at[pl.ds(off_n, CHUNK)], idx[nb])
        pltpu.async_copy(tok_hbm.at[idx[nb]], rows[nb], gsem[nb])

      pltpu.make_async_copy(
          tok_hbm.at[idx[b]], rows[b], gsem[b]).wait()

      @pl.loop(0, SEQ_LEN, unroll=4)
      def _row(r):
        p = [pos_v[r, pl.ds(d * LANES, LANES)] for d in range(D_REGS)]
        for s in range(SEQS_PER_CHUNK):
          rr = s * SEQ_LEN + r
          for d in range(D_REGS):
            sl = pl.ds(d * LANES, LANES)
            rows[b][rr, sl] = rows[b][rr, sl] + p[d]

      pltpu.async_copy(
          rows[b], out_hbm.at[pl.ds(base + i * CHUNK, CHUNK)], wsem[b])

  for b in range(2):
    pltpu.make_async_copy(
        rows[b], out_hbm.at[pl.ds(base, CHUNK)], wsem[b]).wait()


@jax.jit
def _embed(x_flat, token_table, pos_table):
  mesh = plsc.VectorSubcoreMesh(
      core_axis_name="c", subcore_axis_name="s",
      num_cores=NUM_CORES, num_subcores=NUM_SUBCORES,
  )
  run = pl.kernel(
      _embed_body,
      out_type=jax.ShapeDtypeStruct((FLAT, EMBED_DIM), jnp.float32),
      mesh=mesh,
      compiler_params=pltpu.CompilerParams(use_tc_tiling_on_sc=False),
      scratch_types=[
          pltpu.VMEM((CHUNK,), jnp.int32),
          pltpu.VMEM((CHUNK,), jnp.int32),
          pltpu.VMEM((CHUNK, EMBED_DIM), jnp.float32),
          pltpu.VMEM((CHUNK, EMBED_DIM), jnp.float32),
          pltpu.VMEM((SEQ_LEN, EMBED_DIM), jnp.float32),
          pltpu.SemaphoreType.DMA,
          pltpu.SemaphoreType.DMA,
          pltpu.SemaphoreType.DMA,
          pltpu.SemaphoreType.DMA,
      ],
  )
  return run(x_flat, token_table, pos_table)


def kernel(x, token_table, pos_table):
  x_flat = x.reshape(FLAT).astype(jnp.int32)
  out = _embed(x_flat, token_table, pos_table)
  return out.reshape(BATCH, SEQ_LEN, EMBED_DIM)
